# baseline (speedup 1.0000x reference)
import jax
import jax.numpy as jnp
from jax import lax
from jax.experimental import pallas as pl
from jax.experimental.pallas import tpu as pltpu, tpu_sc as plsc

B = 16384
K = 64
NC = 2
NS = 16
L = 16
NW = NC * NS
BPW = B // NW
NBLK = 7813
TPW = 245
ECAP = 768
NCH = B // L
ECH = ECAP // L


def _gather_body(user_hbm, item_hbm, uet_hbm, iet_hbm, uf_hbm, if_hbm,
                 allidx, er, eb, rows_v,
                 blk0, blk1, blk2, sem0, sem1, sem2, sem_w):
    wid = lax.axis_index("s") * NC + lax.axis_index("c")
    lo = wid * 244 + jnp.minimum(wid, 5)
    nb = 244 + jnp.where(wid < 5, 1, 0)
    iota = lax.iota(jnp.int32, L)
    sets = ((blk0, sem0), (blk1, sem1), (blk2, sem2))

    def scalar_at(ref_v, j):
        v = ref_v[pl.ds((j >> 4) * L, L)]
        return jnp.sum(jnp.where(iota == (j & (L - 1)), v, 0))

    def enqueue(t, st, table):
        c = jnp.minimum(lo + t, NBLK - 1)
        off = pl.multiple_of(c << 7, 128)
        pltpu.async_copy(table.at[:, pl.ds(off, 128)], st[0], st[1])

    def phase(idx_hbm, table, outf):
        pltpu.sync_copy(idx_hbm, allidx)
        enqueue(0, sets[0], table)
        enqueue(1, sets[1], table)

        rlo = lo << 7
        rhi = (lo + nb) << 7

        def sel(ch, cnt):
            v = allidx[pl.ds(ch * L, L)]
            m = (v >= rlo) & (v < rhi)
            mi = m.astype(jnp.int32)
            pos = jnp.minimum(cnt + jnp.cumsum(mi) - 1, ECAP - 1)
            plsc.store_scatter(er, [pos], v, mask=m)
            plsc.store_scatter(eb, [pos], ch * L + iota, mask=m)
            return cnt + jnp.sum(mi)

        cnt = lax.fori_loop(0, NCH, sel, jnp.int32(0))

        def extract_entry(e, rv, buf):
            cu = jnp.full((L,), rv & 127, jnp.int32)
            for m in range(K // L):
                rows_v[pl.ds(e * K + m * L, L)] = \
                    plsc.load_gather(buf, [m * L + iota, cu])

        def block_step(t, carry):
            for s in range(3):
                @pl.when(t % 3 == s)
                def _(s=s):
                    @pl.when(t + 2 < TPW)
                    def _():
                        enqueue(t + 2, sets[(s + 2) % 3], table)

                    pltpu.make_async_copy(
                        table.at[:, pl.ds(0, 128)], blk0, sets[s][1]).wait()
                    cc = lo + t

                    def echunk(q, carry2):
                        ech = er[pl.ds(q * L, L)]
                        em = ((ech >> 7) == cc) & ((q * L + iota) < cnt)

                        @pl.when(jnp.any(em))
                        def _():
                            for l in range(L):
                                @pl.when(jnp.any(em & (iota == l)))
                                def _(l=l):
                                    rv = jnp.sum(
                                        jnp.where(iota == l, ech, 0))
                                    extract_entry(q * L + l, rv, sets[s][0])
                        return carry2

                    lax.fori_loop(0, ECH, echunk, 0)
            return carry

        lax.fori_loop(0, TPW, block_step, 0)

        def flush(e, carry):
            be = scalar_at(eb, e)
            pltpu.async_copy(
                rows_v.at[pl.ds(e * K, K)],
                outf.at[pl.ds(be * K, K)], sem_w)
            return carry

        lax.fori_loop(0, cnt, flush, 0)

        def drain(e, carry):
            pltpu.make_async_copy(
                outf.at[pl.ds(0, K)], rows_v.at[pl.ds(0, K)], sem_w).wait()
            return carry

        lax.fori_loop(0, cnt, drain, 0)

    phase(user_hbm, uet_hbm, uf_hbm)
    phase(item_hbm, iet_hbm, if_hbm)


def _dot_body(uf_hbm, if_hbm, out_hbm, uv, iv, out_v):
    wid = lax.axis_index("s") * NC + lax.axis_index("c")
    base = wid * BPW
    pltpu.sync_copy(uf_hbm.at[pl.ds(base * K, BPW * K)], uv)
    pltpu.sync_copy(if_hbm.at[pl.ds(base * K, BPW * K)], iv)
    iota = lax.iota(jnp.int32, L)

    def group(g, carry):
        row_base = (g * L + iota) * K
        acc = jnp.zeros((L,), jnp.float32)
        for k in range(K):
            flat = row_base + ((iota + k) & (K - 1))
            acc = acc + plsc.load_gather(uv, [flat]) * \
                plsc.load_gather(iv, [flat])
        out_v[pl.ds(g * L, L)] = acc
        return carry

    lax.fori_loop(0, BPW // L, group, 0)
    pltpu.sync_copy(out_v, out_hbm.at[pl.ds(base, BPW)])


def kernel(user, item, user_emb, item_emb):
    mesh = plsc.VectorSubcoreMesh(
        core_axis_name="c", subcore_axis_name="s",
        num_cores=NC, num_subcores=NS)
    g1 = pl.kernel(
        _gather_body,
        out_type=(jax.ShapeDtypeStruct((B * K,), jnp.float32),
                  jax.ShapeDtypeStruct((B * K,), jnp.float32)),
        mesh=mesh,
        scratch_types=[
            pltpu.VMEM((B,), jnp.int32),
            pltpu.VMEM((ECAP,), jnp.int32),
            pltpu.VMEM((ECAP,), jnp.int32),
            pltpu.VMEM((ECAP * K,), jnp.float32),
            pltpu.VMEM((K, 128), jnp.float32),
            pltpu.VMEM((K, 128), jnp.float32),
            pltpu.VMEM((K, 128), jnp.float32),
            pltpu.SemaphoreType.DMA,
            pltpu.SemaphoreType.DMA,
            pltpu.SemaphoreType.DMA,
            pltpu.SemaphoreType.DMA,
        ],
        compiler_params=pltpu.CompilerParams(
            needs_layout_passes=False, use_tc_tiling_on_sc=True),
    )
    uf, if_ = g1(user, item, user_emb.T, item_emb.T)
    g2 = pl.kernel(
        _dot_body,
        out_type=jax.ShapeDtypeStruct((B,), jnp.float32),
        mesh=mesh,
        scratch_types=[
            pltpu.VMEM((BPW * K,), jnp.float32),
            pltpu.VMEM((BPW * K,), jnp.float32),
            pltpu.VMEM((BPW,), jnp.float32),
        ],
        compiler_params=pltpu.CompilerParams(needs_layout_passes=False),
    )
    return g2(uf, if_)
